# Initial kernel scaffold; baseline (speedup 1.0000x reference)
import jax
import jax.numpy as jnp
from jax.experimental import pallas as pl


def kernel(t, idx):
    raise NotImplementedError("write your pallas kernel here")


# speedup vs baseline: 1.0161x; 1.0161x over previous
import functools

import jax
import jax.numpy as jnp
from jax import lax
from jax.experimental import pallas as pl
from jax.experimental.pallas import tpu as pltpu
from jax.experimental.pallas import tpu_sc as plsc

B = 16
N = 2048
K = 16384
L = 16
NUM_DMA = 16
K_HALF = K // 2


def _sc_body(a_hbm, eidx_hbm, idx_hbm, out_hbm,
             eidx_v, diag_v, idx_v, out_v, sem):
    c = lax.axis_index("c")
    s = lax.axis_index("s")

    pltpu.sync_copy(eidx_hbm.at[s], eidx_v)
    copies = []
    for j in range(NUM_DMA):
        copies.append(
            pltpu.async_copy(a_hbm.at[eidx_v.at[j]],
                             diag_v.at[pl.ds(128 * j, 128)], sem))
    for cp in copies:
        cp.wait()

    base = c * K_HALF
    pltpu.sync_copy(idx_hbm.at[pl.ds(base, K_HALF)], idx_v)

    def gat(g, carry):
        iv = idx_v[pl.ds(g * L, L)]
        out_v[pl.ds(g * L, L)] = plsc.load_gather(diag_v, [iv])
        return carry
    lax.fori_loop(0, K_HALF // L, gat, 0)

    pltpu.sync_copy(out_v, out_hbm.at[s, pl.ds(base, K_HALF)])


_sc_kernel = functools.partial(
    pl.kernel,
    out_type=jax.ShapeDtypeStruct((B, K), jnp.float32),
    mesh=plsc.VectorSubcoreMesh(core_axis_name="c", subcore_axis_name="s"),
    compiler_params=pltpu.CompilerParams(needs_layout_passes=False),
    scratch_types=[
        pltpu.VMEM((NUM_DMA, 128), jnp.int32),
        pltpu.VMEM((N,), jnp.float32),
        pltpu.VMEM((K_HALF,), jnp.int32),
        pltpu.VMEM((K_HALF,), jnp.float32),
        pltpu.SemaphoreType.DMA,
    ],
)(_sc_body)


def kernel(t, idx):
    a = t.reshape(B * N * N)
    i = jnp.arange(N, dtype=jnp.int32)
    eidx = (jnp.arange(B, dtype=jnp.int32)[:, None] * (N * N)
            + (N + 1) * i[None, :]).reshape(B, NUM_DMA, 128)
    return _sc_kernel(a, eidx, idx.astype(jnp.int32))


# speedup vs baseline: 5.6120x; 5.5231x over previous
import functools

import jax
import jax.numpy as jnp
from jax import lax
from jax.experimental import pallas as pl
from jax.experimental.pallas import tpu as pltpu
from jax.experimental.pallas import tpu_sc as plsc

B = 16
N = 2048
K = 16384
L = 16
BLK = 128
K_HALF = K // 2


def _diag_body(t_ref, out_ref):
    blk = t_ref[...]
    ii = lax.broadcasted_iota(jnp.int32, (B, BLK, BLK), 1)
    jj = lax.broadcasted_iota(jnp.int32, (B, BLK, BLK), 2)
    out_ref[...] = jnp.sum(jnp.where(ii == jj, blk, 0.0), axis=2)


_diag_extract = pl.pallas_call(
    _diag_body,
    grid=(N // BLK,),
    in_specs=[pl.BlockSpec((B, BLK, BLK), lambda k: (0, k, k))],
    out_specs=pl.BlockSpec((B, BLK), lambda k: (0, k)),
    out_shape=jax.ShapeDtypeStruct((B, N), jnp.float32),
)


def _sc_body(diag_hbm, idx_hbm, out_hbm, diag_v, idx_v, out_v):
    c = lax.axis_index("c")
    s = lax.axis_index("s")

    pltpu.sync_copy(diag_hbm.at[pl.ds(s * N, N)], diag_v)
    base = c * K_HALF
    pltpu.sync_copy(idx_hbm.at[pl.ds(base, K_HALF)], idx_v)

    def gat(g, carry):
        iv = idx_v[pl.ds(g * L, L)]
        out_v[pl.ds(g * L, L)] = plsc.load_gather(diag_v, [iv])
        return carry
    lax.fori_loop(0, K_HALF // L, gat, 0)

    pltpu.sync_copy(out_v, out_hbm.at[s, pl.ds(base, K_HALF)])


_sc_lookup = functools.partial(
    pl.kernel,
    out_type=jax.ShapeDtypeStruct((B, K), jnp.float32),
    mesh=plsc.VectorSubcoreMesh(core_axis_name="c", subcore_axis_name="s"),
    compiler_params=pltpu.CompilerParams(needs_layout_passes=False),
    scratch_types=[
        pltpu.VMEM((N,), jnp.float32),
        pltpu.VMEM((K_HALF,), jnp.int32),
        pltpu.VMEM((K_HALF,), jnp.float32),
    ],
)(_sc_body)


def kernel(t, idx):
    diag = _diag_extract(t)
    return _sc_lookup(diag.reshape(B * N), idx.astype(jnp.int32))
